# baseline (speedup 1.0000x reference)
import functools

import jax
import jax.numpy as jnp
from jax import lax
from jax.experimental import pallas as pl
from jax.experimental.pallas import tpu as pltpu
from jax.experimental.pallas import tpu_sc as plsc

_B = 8192
_J = 8
_L = 8
_NPAIR = (_J * (_J - 1)) // 2
_NBIN = _L // 2 + 1
_NCOL = 1 + _J + _NPAIR * _NBIN
_ROWW = _J * _L * _J * _L

_NC = 2
_NS = 16
_NW = _NC * _NS
_RPW = _B // _NW
_CH = 8
_NCHUNK = _RPW // _CH

_PAIR = {}
_p = 0
for _a in range(_J):
    for _b in range(_a + 1, _J):
        _PAIR[(_a, _b)] = _p
        _p += 1


def _tree8(xs):
    return ((xs[0] + xs[1]) + (xs[2] + xs[3])) + ((xs[4] + xs[5]) + (xs[6] + xs[7]))


def _body(s0_hbm, s1_hbm, s2_hbm, out_hbm,
          buf0, buf1, s1c0, s1c1, s0c0, s0c1, stage, sem0, sem1):
    wid = lax.axis_index("s") * _NC + lax.axis_index("c")
    lanes = lax.iota(jnp.int32, 16)
    zeros16 = jnp.zeros((16,), jnp.int32)
    dd = lanes % 8
    half = lanes // 8

    rot = [(lanes // 8) * 8 + ((lanes + l0) % 8) for l0 in range(_L)]
    pfold = (lanes // 8) * 8 + ((8 - lanes % 8) % 8)

    def rotg(v, perm):
        return v.at[perm].get(mode="promise_in_bounds")

    row_base = wid * _RPW

    def fill(c, buf, s1c, s0c, sem):
        r0 = row_base + c * _CH
        pltpu.async_copy(s2_hbm.at[pl.ds(r0, _CH)], buf, sem)
        pltpu.async_copy(s1_hbm.at[pl.ds(r0, _CH)], s1c, sem)
        pltpu.async_copy(s0_hbm.at[pl.ds(r0, _CH)], s0c, sem)

    def drain(c, buf, s1c, s0c, sem):
        r0 = row_base + c * _CH
        pltpu.make_async_copy(s2_hbm.at[pl.ds(r0, _CH)], buf, sem).wait()
        pltpu.make_async_copy(s1_hbm.at[pl.ds(r0, _CH)], s1c, sem).wait()
        pltpu.make_async_copy(s0_hbm.at[pl.ds(r0, _CH)], s0c, sem).wait()

    fill(0, buf0, s1c0, s0c0, sem0)
    fill(1, buf1, s1c1, s0c1, sem1)

    mask8 = lanes < 8
    bl = lanes % 8

    def chunk_pair(cc, carry):
        for bsel, (buf, s1c, s0c, sem) in enumerate(
                ((buf0, s1c0, s0c0, sem0), (buf1, s1c1, s0c1, sem1))):
            c = cc * 2 + bsel
            row0 = row_base + c * _CH
            drain(c, buf, s1c, s0c, sem)

            def batch(i, carry2):
                iv = zeros16 + i
                for j0 in range(_J - 1):
                    for k in range((j0 + 1) // 2, _J // 2):
                        acc = jnp.zeros((16,), jnp.float32)
                        for l0 in range(_L):
                            v = buf[i, j0, pl.ds(l0 * 64 + k * 16, 16)]
                            acc = acc + rotg(v, rot[l0])
                        folded = (acc + rotg(acc, pfold)) * 0.0625
                        lo_ok = 2 * k > j0
                        p1 = _PAIR[(j0, 2 * k + 1)]
                        if lo_ok:
                            p0 = _PAIR[(j0, 2 * k)]
                            cols = (9 + _NBIN * p0) + half * (_NBIN * (p1 - p0)) + dd
                            msk = dd < _NBIN
                        else:
                            cols = (9 + _NBIN * p1) + dd
                            msk = (dd < _NBIN) & (half == 1)
                        plsc.store_scatter(stage, [iv, cols], folded, mask=msk)
                return carry2

            lax.fori_loop(0, _CH, batch, 0)

            for j in range(_J):
                xs = [plsc.load_gather(s1c, [bl, zeros16 + j, zeros16 + l],
                                       mask=mask8)
                      for l in range(_L)]
                plsc.store_scatter(stage, [lanes, zeros16 + (1 + j)],
                                   _tree8(xs) * 0.125, mask=mask8)

            s0v = plsc.load_gather(s0c, [bl, zeros16], mask=mask8)
            plsc.store_scatter(stage, [lanes, zeros16], s0v, mask=mask8)

            pltpu.sync_copy(stage, out_hbm.at[pl.ds(row0, _CH)])

            nxt = c + 2

            @pl.when(nxt < _NCHUNK)
            def _():
                fill(nxt, buf, s1c, s0c, sem)

        return carry

    lax.fori_loop(0, _NCHUNK // 2, chunk_pair, 0)


@jax.jit
def _run(s0, s1, s2):
    mesh = plsc.VectorSubcoreMesh(core_axis_name="c", subcore_axis_name="s")
    return pl.kernel(
        _body,
        out_type=jax.ShapeDtypeStruct((_B, _NCOL), jnp.float32),
        mesh=mesh,
        compiler_params=pltpu.CompilerParams(
            needs_layout_passes=False,
            use_tc_tiling_on_sc=False,
            disable_bounds_checks=True,
        ),
        scratch_types=[
            pltpu.VMEM((_CH, _J, _ROWW // _J), jnp.float32),
            pltpu.VMEM((_CH, _J, _ROWW // _J), jnp.float32),
            pltpu.VMEM((_CH, _J, _L), jnp.float32),
            pltpu.VMEM((_CH, _J, _L), jnp.float32),
            pltpu.VMEM((_CH, 1), jnp.float32),
            pltpu.VMEM((_CH, 1), jnp.float32),
            pltpu.VMEM((_CH, _NCOL), jnp.float32),
            pltpu.SemaphoreType.DMA,
            pltpu.SemaphoreType.DMA,
        ],
    )(s0, s1, s2)


def kernel(s0, s1, s2):
    return _run(s0, s1, s2.reshape(_B, _J, _ROWW // _J))


# speedup vs baseline: 1.3542x; 1.3542x over previous
import functools

import jax
import jax.numpy as jnp
from jax import lax
from jax.experimental import pallas as pl
from jax.experimental.pallas import tpu as pltpu
from jax.experimental.pallas import tpu_sc as plsc

_B = 8192
_J = 8
_L = 8
_NPAIR = (_J * (_J - 1)) // 2
_NBIN = _L // 2 + 1
_NCOL = 1 + _J + _NPAIR * _NBIN
_ROWW = _J * _L * _J * _L

_NC = 2
_NS = 16
_NW = _NC * _NS
_RPW = _B // _NW
_CH = 8
_NCHUNK = _RPW // _CH

_PAIR = {}
_p = 0
for _a in range(_J):
    for _b in range(_a + 1, _J):
        _PAIR[(_a, _b)] = _p
        _p += 1


def _tree8(xs):
    return ((xs[0] + xs[1]) + (xs[2] + xs[3])) + ((xs[4] + xs[5]) + (xs[6] + xs[7]))


def _body(s0_hbm, s1_hbm, s2_hbm, out_hbm,
          buf0, buf1, s1c0, s1c1, s0c0, s0c1, stage, sem0, sem1):
    wid = lax.axis_index("s") * _NC + lax.axis_index("c")
    lanes = lax.iota(jnp.int32, 16)
    zeros16 = jnp.zeros((16,), jnp.int32)
    dd = lanes % 8
    half = lanes // 8

    rot = [(lanes // 8) * 8 + ((lanes + l0) % 8) for l0 in range(_L)]
    pfold = (lanes // 8) * 8 + ((8 - lanes % 8) % 8)

    def rotg(v, perm):
        return v.at[perm].get(mode="promise_in_bounds")

    row_base = wid * _RPW

    def fill(c, buf, s1c, s0c, sem):
        r0 = row_base + c * _CH
        pltpu.async_copy(s2_hbm.at[pl.ds(r0, _CH)], buf, sem)
        pltpu.async_copy(s1_hbm.at[pl.ds(r0, _CH)], s1c, sem)
        pltpu.async_copy(s0_hbm.at[pl.ds(r0, _CH)], s0c, sem)

    def drain(c, buf, s1c, s0c, sem):
        r0 = row_base + c * _CH
        pltpu.make_async_copy(s2_hbm.at[pl.ds(r0, _CH)], buf, sem).wait()
        pltpu.make_async_copy(s1_hbm.at[pl.ds(r0, _CH)], s1c, sem).wait()
        pltpu.make_async_copy(s0_hbm.at[pl.ds(r0, _CH)], s0c, sem).wait()

    fill(0, buf0, s1c0, s0c0, sem0)
    fill(1, buf1, s1c1, s0c1, sem1)

    mask8 = lanes < 8
    bl = lanes % 8

    def chunk_pair(cc, carry):
        for bsel, (buf, s1c, s0c, sem) in enumerate(
                ((buf0, s1c0, s0c0, sem0), (buf1, s1c1, s0c1, sem1))):
            c = cc * 2 + bsel
            row0 = row_base + c * _CH
            drain(c, buf, s1c, s0c, sem)

            def batch(i, carry2):
                iv = zeros16 + i
                for j0 in range(_J - 1):
                    for k in range((j0 + 1) // 2, _J // 2):
                        acc = jnp.zeros((16,), jnp.float32)
                        for l0 in range(_L):
                            v = buf[i, j0, pl.ds(l0 * 64 + k * 16, 16)]
                            acc = acc + rotg(v, rot[l0])
                        folded = (acc + rotg(acc, pfold)) * 0.0625
                        lo_ok = 2 * k > j0
                        p1 = _PAIR[(j0, 2 * k + 1)]
                        if lo_ok:
                            p0 = _PAIR[(j0, 2 * k)]
                            cols = (9 + _NBIN * p0) + half * (_NBIN * (p1 - p0)) + dd
                            msk = dd < _NBIN
                        else:
                            cols = (9 + _NBIN * p1) + dd
                            msk = (dd < _NBIN) & (half == 1)
                        plsc.store_scatter(stage, [iv, cols], folded, mask=msk)
                return carry2

            lax.fori_loop(0, _CH, batch, 0)

            for j in range(_J):
                xs = [plsc.load_gather(s1c, [bl, zeros16 + j, zeros16 + l],
                                       mask=mask8)
                      for l in range(_L)]
                plsc.store_scatter(stage, [lanes, zeros16 + (1 + j)],
                                   _tree8(xs) * 0.125, mask=mask8)

            s0v = plsc.load_gather(s0c, [bl, zeros16], mask=mask8)
            plsc.store_scatter(stage, [lanes, zeros16], s0v, mask=mask8)

            pltpu.sync_copy(stage, out_hbm.at[pl.ds(row0, _CH)])

            nxt = c + 2

            @pl.when(nxt < _NCHUNK)
            def _():
                fill(nxt, buf, s1c, s0c, sem)

        return carry

    lax.fori_loop(0, _NCHUNK // 2, chunk_pair, 0)


@jax.jit
def _run(s0, s1, s2):
    mesh = plsc.VectorSubcoreMesh(core_axis_name="c", subcore_axis_name="s")
    return pl.kernel(
        _body,
        out_type=jax.ShapeDtypeStruct((_B, _NCOL), jnp.float32),
        mesh=mesh,
        compiler_params=pltpu.CompilerParams(
            needs_layout_passes=False,
            disable_bounds_checks=True,
        ),
        scratch_types=[
            pltpu.VMEM((_CH, _J, _ROWW // _J), jnp.float32),
            pltpu.VMEM((_CH, _J, _ROWW // _J), jnp.float32),
            pltpu.VMEM((_CH, _J, _L), jnp.float32),
            pltpu.VMEM((_CH, _J, _L), jnp.float32),
            pltpu.VMEM((_CH, 1), jnp.float32),
            pltpu.VMEM((_CH, 1), jnp.float32),
            pltpu.VMEM((_CH, _NCOL), jnp.float32),
            pltpu.SemaphoreType.DMA,
            pltpu.SemaphoreType.DMA,
        ],
    )(s0, s1, s2)


def kernel(s0, s1, s2):
    return _run(s0, s1, s2.reshape(_B, _J, _ROWW // _J))


# speedup vs baseline: 1.3640x; 1.0073x over previous
import functools

import jax
import jax.numpy as jnp
from jax import lax
from jax.experimental import pallas as pl
from jax.experimental.pallas import tpu as pltpu
from jax.experimental.pallas import tpu_sc as plsc

_B = 8192
_J = 8
_L = 8
_NPAIR = (_J * (_J - 1)) // 2
_NBIN = _L // 2 + 1
_NCOL = 1 + _J + _NPAIR * _NBIN
_ROWW = _J * _L * _J * _L

_NC = 2
_NS = 16
_NW = _NC * _NS
_RPW = _B // _NW
_CH = 8
_NCHUNK = _RPW // _CH

_PAIR = {}
_p = 0
for _a in range(_J):
    for _b in range(_a + 1, _J):
        _PAIR[(_a, _b)] = _p
        _p += 1


def _tree8(xs):
    return ((xs[0] + xs[1]) + (xs[2] + xs[3])) + ((xs[4] + xs[5]) + (xs[6] + xs[7]))


def _body(s0_hbm, s1_hbm, s2_hbm, out_hbm,
          buf0, buf1, s1c0, s1c1, s0c0, s0c1, stage, sem0, sem1):
    wid = lax.axis_index("s") * _NC + lax.axis_index("c")
    lanes = lax.iota(jnp.int32, 16)
    zeros16 = jnp.zeros((16,), jnp.int32)
    dd = lanes % 8
    half = lanes // 8

    rot = [(lanes // 8) * 8 + ((lanes + l0) % 8) for l0 in range(_L)]
    pfold = (lanes // 8) * 8 + ((8 - lanes % 8) % 8)

    def rotg(v, perm):
        return v.at[perm].get(mode="promise_in_bounds")

    row_base = wid * _RPW

    def fill(c, buf, s1c, s0c, sem):
        r0 = row_base + c * _CH
        pltpu.async_copy(s2_hbm.at[pl.ds(r0, _CH)], buf, sem)
        pltpu.async_copy(s1_hbm.at[pl.ds(r0, _CH)], s1c, sem)
        pltpu.async_copy(s0_hbm.at[pl.ds(r0, _CH)], s0c, sem)

    def drain(c, buf, s1c, s0c, sem):
        r0 = row_base + c * _CH
        pltpu.make_async_copy(s2_hbm.at[pl.ds(r0, _CH)], buf, sem).wait()
        pltpu.make_async_copy(s1_hbm.at[pl.ds(r0, _CH)], s1c, sem).wait()
        pltpu.make_async_copy(s0_hbm.at[pl.ds(r0, _CH)], s0c, sem).wait()

    fill(0, buf0, s1c0, s0c0, sem0)
    fill(1, buf1, s1c1, s0c1, sem1)

    mask8 = lanes < 8
    bl = lanes % 8

    def chunk_pair(cc, carry):
        for bsel, (buf, s1c, s0c, sem) in enumerate(
                ((buf0, s1c0, s0c0, sem0), (buf1, s1c1, s0c1, sem1))):
            c = cc * 2 + bsel
            row0 = row_base + c * _CH
            drain(c, buf, s1c, s0c, sem)

            def batch(i, carry2):
                iv = zeros16 + i
                for j0 in range(_J - 1):
                    for k in range((j0 + 1) // 2, _J // 2):
                        acc = jnp.zeros((16,), jnp.float32)
                        for l0 in range(_L):
                            v = buf[i, pl.ds(j0 * 512 + l0 * 64 + k * 16, 16)]
                            acc = acc + rotg(v, rot[l0])
                        folded = (acc + rotg(acc, pfold)) * 0.0625
                        lo_ok = 2 * k > j0
                        p1 = _PAIR[(j0, 2 * k + 1)]
                        if lo_ok:
                            p0 = _PAIR[(j0, 2 * k)]
                            cols = (9 + _NBIN * p0) + half * (_NBIN * (p1 - p0)) + dd
                            msk = dd < _NBIN
                        else:
                            cols = (9 + _NBIN * p1) + dd
                            msk = (dd < _NBIN) & (half == 1)
                        plsc.store_scatter(stage, [iv, cols], folded, mask=msk)
                return carry2

            lax.fori_loop(0, _CH, batch, 0)

            for j in range(_J):
                xs = [plsc.load_gather(s1c, [bl, zeros16 + j, zeros16 + l],
                                       mask=mask8)
                      for l in range(_L)]
                plsc.store_scatter(stage, [lanes, zeros16 + (1 + j)],
                                   _tree8(xs) * 0.125, mask=mask8)

            s0v = plsc.load_gather(s0c, [bl, zeros16], mask=mask8)
            plsc.store_scatter(stage, [lanes, zeros16], s0v, mask=mask8)

            pltpu.sync_copy(stage, out_hbm.at[pl.ds(row0, _CH)])

            nxt = c + 2

            @pl.when(nxt < _NCHUNK)
            def _():
                fill(nxt, buf, s1c, s0c, sem)

        return carry

    lax.fori_loop(0, _NCHUNK // 2, chunk_pair, 0)


@jax.jit
def _run(s0, s1, s2):
    mesh = plsc.VectorSubcoreMesh(core_axis_name="c", subcore_axis_name="s")
    return pl.kernel(
        _body,
        out_type=jax.ShapeDtypeStruct((_B, _NCOL), jnp.float32),
        mesh=mesh,
        compiler_params=pltpu.CompilerParams(needs_layout_passes=False),
        scratch_types=[
            pltpu.VMEM((_CH, _ROWW), jnp.float32),
            pltpu.VMEM((_CH, _ROWW), jnp.float32),
            pltpu.VMEM((_CH, _J, _L), jnp.float32),
            pltpu.VMEM((_CH, _J, _L), jnp.float32),
            pltpu.VMEM((_CH, 1), jnp.float32),
            pltpu.VMEM((_CH, 1), jnp.float32),
            pltpu.VMEM((_CH, _NCOL), jnp.float32),
            pltpu.SemaphoreType.DMA,
            pltpu.SemaphoreType.DMA,
        ],
    )(s0, s1, s2)


def kernel(s0, s1, s2):
    return _run(s0, s1, s2.reshape(_B, _ROWW))
